# baseline (speedup 1.0000x reference)
import jax
import jax.numpy as jnp
from jax.experimental import pallas as pl
from jax.experimental.pallas import tpu as pltpu

B, Q, L = 32, 300, 256
P = 256
QP = 384


def _stats_body(logits_ref, boxes_ref, scale_ref, rowmax_ref, denom_ref, xyxy_ref):
    x = logits_ref[0]
    rm = jnp.max(x, axis=-1, keepdims=True)
    rowmax_ref[0] = rm
    denom_ref[0] = jnp.sum(jnp.exp(x - rm), axis=-1, keepdims=True)
    bx = boxes_ref[0]
    cx, cy, w, h = bx[:, 0:1], bx[:, 1:2], bx[:, 2:3], bx[:, 3:4]
    xyxy = jnp.concatenate(
        [cx - 0.5 * w, cy - 0.5 * h, cx + 0.5 * w, cy + 0.5 * h], axis=1)
    xyxy_ref[0] = xyxy * scale_ref[0]


def _phrase_body(idx_ref, logits_ref, pos_ref, rowmax_ref, denom_ref,
                 xyxy_ref, eye_ref, tri_ref, ob_ref, os_ref):
    x = logits_ref[0]
    pos = pos_ref[0] > 0
    m = jnp.max(jnp.where(pos, x, -jnp.inf), axis=-1, keepdims=True)
    s = jnp.exp(m - rowmax_ref[0]) / denom_ref[0]
    row_iota = jax.lax.broadcasted_iota(jnp.int32, (QP, 1), 0)
    s = jnp.where(row_iota < Q, s, -1.0)

    eye = eye_ref[...]
    s_row = jnp.sum(eye * s, axis=0, keepdims=True)

    tri = tri_ref[...]
    ind = (jnp.where(s > s_row, 1.0, 0.0)
           + jnp.where(s == s_row, tri, 0.0))
    rank = jnp.sum(ind, axis=0, keepdims=True)

    o_col = jax.lax.broadcasted_iota(jnp.int32, (QP, 1), 0).astype(jnp.float32)
    perm = (rank == o_col).astype(jnp.float32)
    payload = jnp.concatenate([xyxy_ref[0], s], axis=1)
    res = jnp.dot(perm, payload, preferred_element_type=jnp.float32)
    ob_ref[0] = res[:, 0:4]
    os_ref[0] = res[:, 4:5]


def kernel(pred_logits, pred_boxes, target_sizes, positive_map, phrase_batch_idx):
    ts = target_sizes.astype(jnp.float32)
    scale = jnp.stack([ts[:, 1], ts[:, 0], ts[:, 1], ts[:, 0]], axis=1)
    scale = scale.reshape(B, 1, 4)
    pos3 = positive_map.reshape(P, 1, L)
    logits_p = jnp.pad(pred_logits, ((0, 0), (0, QP - Q), (0, 0)))
    boxes_p = jnp.pad(pred_boxes, ((0, 0), (0, QP - Q), (0, 0)))
    qi = jnp.arange(QP)
    eye = (qi[:, None] == qi[None, :]).astype(jnp.float32)
    tri = (qi[:, None] < qi[None, :]).astype(jnp.float32)

    rowmax, denom, xyxy = pl.pallas_call(
        _stats_body,
        grid=(B,),
        in_specs=[
            pl.BlockSpec((1, QP, L), lambda b: (b, 0, 0)),
            pl.BlockSpec((1, QP, 4), lambda b: (b, 0, 0)),
            pl.BlockSpec((1, 1, 4), lambda b: (b, 0, 0)),
        ],
        out_specs=[
            pl.BlockSpec((1, QP, 1), lambda b: (b, 0, 0)),
            pl.BlockSpec((1, QP, 1), lambda b: (b, 0, 0)),
            pl.BlockSpec((1, QP, 4), lambda b: (b, 0, 0)),
        ],
        out_shape=[
            jax.ShapeDtypeStruct((B, QP, 1), jnp.float32),
            jax.ShapeDtypeStruct((B, QP, 1), jnp.float32),
            jax.ShapeDtypeStruct((B, QP, 4), jnp.float32),
        ],
        compiler_params=pltpu.CompilerParams(
            dimension_semantics=("arbitrary",),
        ),
    )(logits_p, boxes_p, scale)

    grid_spec = pltpu.PrefetchScalarGridSpec(
        num_scalar_prefetch=1,
        grid=(P,),
        in_specs=[
            pl.BlockSpec((1, QP, L), lambda p, idx: (idx[p], 0, 0)),
            pl.BlockSpec((1, 1, L), lambda p, idx: (p, 0, 0)),
            pl.BlockSpec((1, QP, 1), lambda p, idx: (idx[p], 0, 0)),
            pl.BlockSpec((1, QP, 1), lambda p, idx: (idx[p], 0, 0)),
            pl.BlockSpec((1, QP, 4), lambda p, idx: (idx[p], 0, 0)),
            pl.BlockSpec((QP, QP), lambda p, idx: (0, 0)),
            pl.BlockSpec((QP, QP), lambda p, idx: (0, 0)),
        ],
        out_specs=[
            pl.BlockSpec((1, QP, 4), lambda p, idx: (p, 0, 0)),
            pl.BlockSpec((1, QP, 1), lambda p, idx: (p, 0, 0)),
        ],
    )
    sorted_boxes, sorted_scores = pl.pallas_call(
        _phrase_body,
        grid_spec=grid_spec,
        out_shape=[
            jax.ShapeDtypeStruct((P, QP, 4), jnp.float32),
            jax.ShapeDtypeStruct((P, QP, 1), jnp.float32),
        ],
        compiler_params=pltpu.CompilerParams(
            dimension_semantics=("arbitrary",),
        ),
    )(phrase_batch_idx, logits_p, pos3, rowmax, denom, xyxy, eye, tri)
    return (sorted_boxes[:, :Q, :], sorted_scores[:, :Q, 0])


# speedup vs baseline: 1.2977x; 1.2977x over previous
import jax
import jax.numpy as jnp
from jax.experimental import pallas as pl
from jax.experimental.pallas import tpu as pltpu

B, Q, L = 32, 300, 256
P = 256


def _stats_body(logits_ref, boxes_ref, scale_ref, rowmax_ref, denom_ref, xyxy_ref):
    x = logits_ref[0]
    rm = jnp.max(x, axis=-1, keepdims=True)
    rowmax_ref[0] = rm
    denom_ref[0] = jnp.sum(jnp.exp(x - rm), axis=-1, keepdims=True)
    bx = boxes_ref[0]
    cx, cy, w, h = bx[:, 0:1], bx[:, 1:2], bx[:, 2:3], bx[:, 3:4]
    xyxy = jnp.concatenate(
        [cx - 0.5 * w, cy - 0.5 * h, cx + 0.5 * w, cy + 0.5 * h], axis=1)
    xyxy_ref[0] = xyxy * scale_ref[0]


def _phrase_body(idx_ref, logits_ref, pos_ref, rowmax_ref, denom_ref,
                 xyxy_ref, eye_ref, tri_ref, ob_ref, os_ref):
    x = logits_ref[0]
    pos = pos_ref[0] > 0
    m = jnp.max(jnp.where(pos, x, -jnp.inf), axis=-1, keepdims=True)
    s = jnp.exp(m - rowmax_ref[0]) / denom_ref[0]

    eye = eye_ref[...]
    s_row = jnp.sum(eye * s, axis=0, keepdims=True)

    tri = tri_ref[...]
    ind = (jnp.where(s > s_row, 1.0, 0.0)
           + jnp.where(s == s_row, tri, 0.0))
    rank = jnp.sum(ind, axis=0, keepdims=True)

    o_col = jax.lax.broadcasted_iota(jnp.int32, (Q, 1), 0).astype(jnp.float32)
    perm = (rank == o_col).astype(jnp.float32)
    payload = jnp.concatenate([xyxy_ref[0], s], axis=1)
    res = jnp.dot(perm, payload, preferred_element_type=jnp.float32)
    ob_ref[0] = res[:, 0:4]
    os_ref[0] = res[:, 4:5]


def kernel(pred_logits, pred_boxes, target_sizes, positive_map, phrase_batch_idx):
    ts = target_sizes.astype(jnp.float32)
    scale = jnp.stack([ts[:, 1], ts[:, 0], ts[:, 1], ts[:, 0]], axis=1)
    scale = scale.reshape(B, 1, 4)
    pos3 = positive_map.reshape(P, 1, L)
    qi = jnp.arange(Q)
    eye = (qi[:, None] == qi[None, :]).astype(jnp.float32)
    tri = (qi[:, None] < qi[None, :]).astype(jnp.float32)

    rowmax, denom, xyxy = pl.pallas_call(
        _stats_body,
        grid=(B,),
        in_specs=[
            pl.BlockSpec((1, Q, L), lambda b: (b, 0, 0)),
            pl.BlockSpec((1, Q, 4), lambda b: (b, 0, 0)),
            pl.BlockSpec((1, 1, 4), lambda b: (b, 0, 0)),
        ],
        out_specs=[
            pl.BlockSpec((1, Q, 1), lambda b: (b, 0, 0)),
            pl.BlockSpec((1, Q, 1), lambda b: (b, 0, 0)),
            pl.BlockSpec((1, Q, 4), lambda b: (b, 0, 0)),
        ],
        out_shape=[
            jax.ShapeDtypeStruct((B, Q, 1), jnp.float32),
            jax.ShapeDtypeStruct((B, Q, 1), jnp.float32),
            jax.ShapeDtypeStruct((B, Q, 4), jnp.float32),
        ],
        compiler_params=pltpu.CompilerParams(
            dimension_semantics=("arbitrary",),
        ),
    )(pred_logits, pred_boxes, scale)

    grid_spec = pltpu.PrefetchScalarGridSpec(
        num_scalar_prefetch=1,
        grid=(P,),
        in_specs=[
            pl.BlockSpec((1, Q, L), lambda p, idx: (idx[p], 0, 0)),
            pl.BlockSpec((1, 1, L), lambda p, idx: (p, 0, 0)),
            pl.BlockSpec((1, Q, 1), lambda p, idx: (idx[p], 0, 0)),
            pl.BlockSpec((1, Q, 1), lambda p, idx: (idx[p], 0, 0)),
            pl.BlockSpec((1, Q, 4), lambda p, idx: (idx[p], 0, 0)),
            pl.BlockSpec((Q, Q), lambda p, idx: (0, 0)),
            pl.BlockSpec((Q, Q), lambda p, idx: (0, 0)),
        ],
        out_specs=[
            pl.BlockSpec((1, Q, 4), lambda p, idx: (p, 0, 0)),
            pl.BlockSpec((1, Q, 1), lambda p, idx: (p, 0, 0)),
        ],
    )
    sorted_boxes, sorted_scores = pl.pallas_call(
        _phrase_body,
        grid_spec=grid_spec,
        out_shape=[
            jax.ShapeDtypeStruct((P, Q, 4), jnp.float32),
            jax.ShapeDtypeStruct((P, Q, 1), jnp.float32),
        ],
        compiler_params=pltpu.CompilerParams(
            dimension_semantics=("arbitrary",),
        ),
    )(phrase_batch_idx, pred_logits, pos3, rowmax, denom, xyxy, eye, tri)
    return (sorted_boxes, sorted_scores.reshape(P, Q))


# speedup vs baseline: 1.3070x; 1.0072x over previous
import jax
import jax.numpy as jnp
import numpy as np
from jax.experimental import pallas as pl
from jax.experimental.pallas import tpu as pltpu

B, Q, L = 32, 300, 256
P = 256

_QI = np.arange(Q)
_EYE = (_QI[:, None] == _QI[None, :]).astype(np.float32)
_TRI = (_QI[:, None] < _QI[None, :]).astype(np.float32)


def _stats_body(logits_ref, boxes_ref, scale_ref, rowmax_ref, denom_ref, xyxy_ref):
    x = logits_ref[0]
    rm = jnp.max(x, axis=-1, keepdims=True)
    rowmax_ref[0] = rm
    denom_ref[0] = jnp.sum(jnp.exp(x - rm), axis=-1, keepdims=True)
    bx = boxes_ref[0]
    cx, cy, w, h = bx[:, 0:1], bx[:, 1:2], bx[:, 2:3], bx[:, 3:4]
    xyxy = jnp.concatenate(
        [cx - 0.5 * w, cy - 0.5 * h, cx + 0.5 * w, cy + 0.5 * h], axis=1)
    xyxy_ref[0] = xyxy * scale_ref[0]


def _phrase_body(idx_ref, logits_ref, pos_ref, rowmax_ref, denom_ref,
                 xyxy_ref, eye_ref, tri_ref, ob_ref, os_ref):
    x = logits_ref[0]
    pos = pos_ref[0] > 0
    m = jnp.max(jnp.where(pos, x, -jnp.inf), axis=-1, keepdims=True)
    s = jnp.exp(m - rowmax_ref[0]) / denom_ref[0]

    eye = eye_ref[...]
    s_row = jnp.sum(eye * s, axis=0, keepdims=True)

    tri = tri_ref[...]
    ind = (jnp.where(s > s_row, 1.0, 0.0)
           + jnp.where(s == s_row, tri, 0.0))
    rank = jnp.sum(ind, axis=0, keepdims=True)

    o_col = jax.lax.broadcasted_iota(jnp.int32, (Q, 1), 0).astype(jnp.float32)
    perm = (rank == o_col).astype(jnp.float32)
    payload = jnp.concatenate([xyxy_ref[0], s], axis=1)
    res = jnp.dot(perm, payload, preferred_element_type=jnp.float32)
    ob_ref[0] = res[:, 0:4]
    os_ref[0] = res[:, 4:5]


def kernel(pred_logits, pred_boxes, target_sizes, positive_map, phrase_batch_idx):
    ts = target_sizes.astype(jnp.float32)
    scale = jnp.stack([ts[:, 1], ts[:, 0], ts[:, 1], ts[:, 0]], axis=1)
    scale = scale.reshape(B, 1, 4)
    pos3 = positive_map.reshape(P, 1, L)
    eye = jnp.asarray(_EYE)
    tri = jnp.asarray(_TRI)

    rowmax, denom, xyxy = pl.pallas_call(
        _stats_body,
        grid=(B,),
        in_specs=[
            pl.BlockSpec((1, Q, L), lambda b: (b, 0, 0)),
            pl.BlockSpec((1, Q, 4), lambda b: (b, 0, 0)),
            pl.BlockSpec((1, 1, 4), lambda b: (b, 0, 0)),
        ],
        out_specs=[
            pl.BlockSpec((1, Q, 1), lambda b: (b, 0, 0)),
            pl.BlockSpec((1, Q, 1), lambda b: (b, 0, 0)),
            pl.BlockSpec((1, Q, 4), lambda b: (b, 0, 0)),
        ],
        out_shape=[
            jax.ShapeDtypeStruct((B, Q, 1), jnp.float32),
            jax.ShapeDtypeStruct((B, Q, 1), jnp.float32),
            jax.ShapeDtypeStruct((B, Q, 4), jnp.float32),
        ],
        compiler_params=pltpu.CompilerParams(
            dimension_semantics=("arbitrary",),
        ),
    )(pred_logits, pred_boxes, scale)

    grid_spec = pltpu.PrefetchScalarGridSpec(
        num_scalar_prefetch=1,
        grid=(P,),
        in_specs=[
            pl.BlockSpec((1, Q, L), lambda p, idx: (idx[p], 0, 0)),
            pl.BlockSpec((1, 1, L), lambda p, idx: (p, 0, 0)),
            pl.BlockSpec((1, Q, 1), lambda p, idx: (idx[p], 0, 0)),
            pl.BlockSpec((1, Q, 1), lambda p, idx: (idx[p], 0, 0)),
            pl.BlockSpec((1, Q, 4), lambda p, idx: (idx[p], 0, 0)),
            pl.BlockSpec((Q, Q), lambda p, idx: (0, 0)),
            pl.BlockSpec((Q, Q), lambda p, idx: (0, 0)),
        ],
        out_specs=[
            pl.BlockSpec((1, Q, 4), lambda p, idx: (p, 0, 0)),
            pl.BlockSpec((1, Q, 1), lambda p, idx: (p, 0, 0)),
        ],
    )
    sorted_boxes, sorted_scores = pl.pallas_call(
        _phrase_body,
        grid_spec=grid_spec,
        out_shape=[
            jax.ShapeDtypeStruct((P, Q, 4), jnp.float32),
            jax.ShapeDtypeStruct((P, Q, 1), jnp.float32),
        ],
        compiler_params=pltpu.CompilerParams(
            dimension_semantics=("arbitrary",),
        ),
    )(phrase_batch_idx, pred_logits, pos3, rowmax, denom, xyxy, eye, tri)
    return (sorted_boxes, sorted_scores.reshape(P, Q))


# speedup vs baseline: 2.4086x; 1.8428x over previous
import jax
import jax.numpy as jnp
import numpy as np
from jax.experimental import pallas as pl
from jax.experimental.pallas import tpu as pltpu

B, Q, L = 32, 300, 256
P = 256

_QI = np.arange(Q)
_EYE = (_QI[:, None] == _QI[None, :]).astype(np.float32)
_TRIL = (_QI[:, None] > _QI[None, :]).astype(np.float32)


def _body(starts_ref, logits_ref, boxesT_ref, scale_ref, pos_ref,
          eye_ref, tril_ref, ob_ref, os_ref):
    b = pl.program_id(0)
    x = logits_ref[0]
    rm = jnp.max(x, axis=-1, keepdims=True)
    denom = jnp.sum(jnp.exp(x - rm), axis=-1, keepdims=True)
    bxT = boxesT_ref[0]
    cx, cy, w, h = bxT[0:1], bxT[1:2], bxT[2:3], bxT[3:4]
    xyxyT = jnp.concatenate(
        [cx - 0.5 * w, cy - 0.5 * h, cx + 0.5 * w, cy + 0.5 * h], axis=0)
    xyxyT = xyxyT * scale_ref[0]
    eye = eye_ref[...]
    tril = tril_ref[...]
    o_row = jax.lax.broadcasted_iota(jnp.int32, (1, Q), 1).astype(jnp.float32)

    def one_phrase(p, carry):
        pos = pos_ref[pl.ds(p, 1), 0, :] > 0.0
        m = jnp.max(jnp.where(pos, x, -jnp.inf), axis=-1, keepdims=True)
        s = jnp.exp(m - rm) / denom

        s_row = jnp.sum(eye * s, axis=0, keepdims=True)

        indT = (jnp.where(s_row > s, 1.0, 0.0)
                + jnp.where(s_row == s, tril, 0.0))
        rank = jnp.sum(indT, axis=1, keepdims=True)

        permT = (rank == o_row).astype(jnp.float32)
        payloadT = jnp.concatenate([xyxyT, s_row], axis=0)
        resT = jnp.dot(payloadT, permT, preferred_element_type=jnp.float32)
        ob_ref[pl.ds(p, 1), :, :] = resT[0:4].reshape(1, 4, Q)
        os_ref[pl.ds(p, 1), :, :] = resT[4:5].reshape(1, 1, Q)
        return carry

    jax.lax.fori_loop(starts_ref[b], starts_ref[b + 1], one_phrase, 0)


def kernel(pred_logits, pred_boxes, target_sizes, positive_map, phrase_batch_idx):
    ts = target_sizes.astype(jnp.float32)
    scale = jnp.stack([ts[:, 1], ts[:, 0], ts[:, 1], ts[:, 0]], axis=1)
    scale = scale.reshape(B, 4, 1)
    boxesT = jnp.transpose(pred_boxes, (0, 2, 1))
    posf = (positive_map > 0).astype(jnp.float32).reshape(P, 1, L)
    starts = jnp.searchsorted(phrase_batch_idx, jnp.arange(B + 1)).astype(jnp.int32)
    eye = jnp.asarray(_EYE)
    tril = jnp.asarray(_TRIL)

    grid_spec = pltpu.PrefetchScalarGridSpec(
        num_scalar_prefetch=1,
        grid=(B,),
        in_specs=[
            pl.BlockSpec((1, Q, L), lambda b, starts: (b, 0, 0)),
            pl.BlockSpec((1, 4, Q), lambda b, starts: (b, 0, 0)),
            pl.BlockSpec((1, 4, 1), lambda b, starts: (b, 0, 0)),
            pl.BlockSpec((P, 1, L), lambda b, starts: (0, 0, 0)),
            pl.BlockSpec((Q, Q), lambda b, starts: (0, 0)),
            pl.BlockSpec((Q, Q), lambda b, starts: (0, 0)),
        ],
        out_specs=[
            pl.BlockSpec((P, 4, Q), lambda b, starts: (0, 0, 0)),
            pl.BlockSpec((P, 1, Q), lambda b, starts: (0, 0, 0)),
        ],
    )
    ob, os_ = pl.pallas_call(
        _body,
        grid_spec=grid_spec,
        out_shape=[
            jax.ShapeDtypeStruct((P, 4, Q), jnp.float32),
            jax.ShapeDtypeStruct((P, 1, Q), jnp.float32),
        ],
        compiler_params=pltpu.CompilerParams(
            dimension_semantics=("arbitrary",),
        ),
    )(starts, pred_logits, boxesT, scale, posf, eye, tril)
    return (jnp.transpose(ob, (0, 2, 1)), os_.reshape(P, Q))


# speedup vs baseline: 3.1184x; 1.2947x over previous
import jax
import jax.numpy as jnp
import numpy as np
from jax.experimental import pallas as pl
from jax.experimental.pallas import tpu as pltpu

B, Q, L = 32, 300, 256
P = 256

_QI = np.arange(Q)
_EYE = (_QI[:, None] == _QI[None, :]).astype(np.float32)
_TRIL = (_QI[:, None] > _QI[None, :]).astype(np.float32)


def _body(starts_ref, logits_ref, boxesT_ref, scale_ref, pos_ref,
          eye_ref, tril_ref, ob_ref, os_ref):
    b = pl.program_id(0)
    x = logits_ref[0]
    rm = jnp.max(x, axis=-1, keepdims=True)
    denom = jnp.sum(jnp.exp(x - rm), axis=-1, keepdims=True)
    bxT = boxesT_ref[0]
    cx, cy, w, h = bxT[0:1], bxT[1:2], bxT[2:3], bxT[3:4]
    xyxyT = jnp.concatenate(
        [cx - 0.5 * w, cy - 0.5 * h, cx + 0.5 * w, cy + 0.5 * h], axis=0)
    xyxyT = xyxyT * scale_ref[0]
    eye = eye_ref[...]
    tril = tril_ref[...]
    o_row = jax.lax.broadcasted_iota(jnp.int32, (1, Q), 1).astype(jnp.float32)

    def compute(p):
        pos = pos_ref[pl.ds(p, 1), 0, :] > 0.0
        m = jnp.max(jnp.where(pos, x, -jnp.inf), axis=-1, keepdims=True)
        s = jnp.exp(m - rm) / denom

        s_row = jnp.sum(eye * s, axis=0, keepdims=True)

        indT = (jnp.where(s_row > s, 1.0, 0.0)
                + jnp.where(s_row == s, tril, 0.0))
        rank = jnp.sum(indT, axis=1, keepdims=True)

        permT = (rank == o_row).astype(jnp.float32)
        payloadT = jnp.concatenate([xyxyT, s_row], axis=0)
        return jnp.dot(payloadT, permT, preferred_element_type=jnp.float32)

    def write(p, resT):
        ob_ref[pl.ds(p, 1), :, :] = resT[0:4].reshape(1, 4, Q)
        os_ref[pl.ds(p, 1), :, :] = resT[4:5].reshape(1, 1, Q)

    start = starts_ref[b]
    end = starts_ref[b + 1]

    def pair(j, carry):
        p0 = start + 2 * j
        p1 = p0 + 1
        p1c = jnp.minimum(p1, jnp.maximum(end - 1, start))
        resT0 = compute(p0)
        resT1 = compute(p1c)
        write(p0, resT0)

        @pl.when(p1 < end)
        def _():
            write(p1, resT1)
        return carry

    jax.lax.fori_loop(jnp.int32(0), (end - start + 1) // 2, pair, 0)


def kernel(pred_logits, pred_boxes, target_sizes, positive_map, phrase_batch_idx):
    ts = target_sizes.astype(jnp.float32)
    scale = jnp.stack([ts[:, 1], ts[:, 0], ts[:, 1], ts[:, 0]], axis=1)
    scale = scale.reshape(B, 4, 1)
    boxesT = jnp.transpose(pred_boxes, (0, 2, 1))
    posf = (positive_map > 0).astype(jnp.float32).reshape(P, 1, L)
    starts = jnp.searchsorted(phrase_batch_idx, jnp.arange(B + 1)).astype(jnp.int32)
    eye = jnp.asarray(_EYE)
    tril = jnp.asarray(_TRIL)

    grid_spec = pltpu.PrefetchScalarGridSpec(
        num_scalar_prefetch=1,
        grid=(B,),
        in_specs=[
            pl.BlockSpec((1, Q, L), lambda b, starts: (b, 0, 0)),
            pl.BlockSpec((1, 4, Q), lambda b, starts: (b, 0, 0)),
            pl.BlockSpec((1, 4, 1), lambda b, starts: (b, 0, 0)),
            pl.BlockSpec((P, 1, L), lambda b, starts: (0, 0, 0)),
            pl.BlockSpec((Q, Q), lambda b, starts: (0, 0)),
            pl.BlockSpec((Q, Q), lambda b, starts: (0, 0)),
        ],
        out_specs=[
            pl.BlockSpec((P, 4, Q), lambda b, starts: (0, 0, 0)),
            pl.BlockSpec((P, 1, Q), lambda b, starts: (0, 0, 0)),
        ],
    )
    ob, os_ = pl.pallas_call(
        _body,
        grid_spec=grid_spec,
        out_shape=[
            jax.ShapeDtypeStruct((P, 4, Q), jnp.float32),
            jax.ShapeDtypeStruct((P, 1, Q), jnp.float32),
        ],
        compiler_params=pltpu.CompilerParams(
            dimension_semantics=("arbitrary",),
        ),
    )(starts, pred_logits, boxesT, scale, posf, eye, tril)
    return (jnp.transpose(ob, (0, 2, 1)), os_.reshape(P, Q))
